# baseline (speedup 1.0000x reference)
import dataclasses
import functools

import jax
import jax.numpy as jnp
from jax import lax
from jax.experimental import pallas as pl
from jax.experimental.pallas import tpu as pltpu
from jax.experimental.pallas import tpu_sc as plsc

B = 16384
D = 64
NC = 2
NS = 16
NW = NC * NS
BPW = B // NW
CH = 64
NCHUNK = BPW // CH
L = 16
G = 16
NG = CH // G


def _sqrt16(x):
    x = jnp.maximum(x, jnp.float32(1e-30))
    xi = lax.bitcast_convert_type(x, jnp.int32)
    yi = (xi >> 1) + jnp.int32(0x1FBD1DF5)
    y = lax.bitcast_convert_type(yi, jnp.float32)
    half = jnp.float32(0.5)
    y = half * (y + x / y)
    y = half * (y + x / y)
    y = half * (y + x / y)
    return y


def _make_sc_kernel():
    mesh = plsc.VectorSubcoreMesh(core_axis_name="c", subcore_axis_name="s")
    out_sds = jax.ShapeDtypeStruct((B,), jnp.float32)
    cp = pltpu.CompilerParams()
    if "needs_layout_passes" in pltpu.CompilerParams.__dataclass_fields__:
        cp = dataclasses.replace(cp, needs_layout_passes=False)
    if "use_tc_tiling_on_sc" in pltpu.CompilerParams.__dataclass_fields__:
        cp = dataclasses.replace(cp, use_tc_tiling_on_sc=True)

    @functools.partial(
        pl.kernel,
        mesh=mesh,
        out_type=(out_sds, out_sds),
        compiler_params=cp,
        scratch_types=[
            pltpu.VMEM((CH,), jnp.int32),
            pltpu.VMEM((CH,), jnp.int32),
            pltpu.VMEM((CH,), jnp.int32),
            pltpu.VMEM((CH,), jnp.int32),
            pltpu.VMEM((CH,), jnp.int32),
            pltpu.VMEM((CH, D), jnp.float32),
            pltpu.VMEM((CH, D), jnp.float32),
            pltpu.VMEM((CH, D), jnp.float32),
            pltpu.VMEM((CH, D), jnp.float32),
            pltpu.VMEM((CH, D), jnp.float32),
            pltpu.VMEM((L, L), jnp.float32),
            pltpu.VMEM((L, L), jnp.float32),
            pltpu.VMEM((CH,), jnp.float32),
            pltpu.VMEM((CH,), jnp.float32),
            pltpu.SemaphoreType.DMA,
            pltpu.SemaphoreType.DMA,
            pltpu.SemaphoreType.DMA,
            pltpu.SemaphoreType.DMA,
        ],
    )
    def trans_e(ph_hbm, pt_hbm, nh_hbm, nt_hbm, et_hbm, ent_hbm, rel_hbm,
                pos_out, neg_out,
                phv, ptv, nhv, ntv, etv, hv, tv, gv, uv, rv,
                mp, mn, psv, nsv, sem0, sem1, sem2, sem3):
        wid = lax.axis_index("s") * NC + lax.axis_index("c")
        row_iota = lax.iota(jnp.int32, L)
        sems = (sem0, sem1, sem2, sem3)

        @pl.loop(0, NCHUNK)
        def _(c):
            base = wid * BPW + c * CH
            pltpu.sync_copy(ph_hbm.at[pl.ds(base, CH)], phv)
            pltpu.sync_copy(pt_hbm.at[pl.ds(base, CH)], ptv)
            pltpu.sync_copy(nh_hbm.at[pl.ds(base, CH)], nhv)
            pltpu.sync_copy(nt_hbm.at[pl.ds(base, CH)], ntv)
            pltpu.sync_copy(et_hbm.at[pl.ds(base, CH)], etv)

            group_cps = []
            for gi in range(NG):
                g = gi * G
                cps = []
                for (iv, buf, table) in ((phv, hv, ent_hbm), (ptv, tv, ent_hbm),
                                         (nhv, gv, ent_hbm), (ntv, uv, ent_hbm),
                                         (etv, rv, rel_hbm)):
                    ivec = iv[pl.ds(g, G)]
                    for rr in range(G):
                        cps.append(pltpu.async_copy(
                            table.at[ivec[rr]], buf.at[g + rr], sems[gi]))
                group_cps.append(cps)

            for gi in range(NG):
                g = gi * G
                for cp_ in group_cps[gi]:
                    cp_.wait()
                for rr in range(G):
                    row = g + rr
                    pacc = None
                    nacc = None
                    for j in range(D // L):
                        sl = pl.ds(j * L, L)
                        rel16 = rv[row, sl]
                        dp = hv[row, sl] + rel16 - tv[row, sl]
                        dn = gv[row, sl] + rel16 - uv[row, sl]
                        pacc = dp * dp if pacc is None else pacc + dp * dp
                        nacc = dn * dn if nacc is None else nacc + dn * dn
                    mp[rr, pl.ds(0, L)] = pacc
                    mn[rr, pl.ds(0, L)] = nacc
                psum = None
                nsum = None
                for col in range(L):
                    ci = jnp.full((L,), col, jnp.int32)
                    pc = plsc.load_gather(mp, [row_iota, ci])
                    ncol = plsc.load_gather(mn, [row_iota, ci])
                    psum = pc if psum is None else psum + pc
                    nsum = ncol if nsum is None else nsum + ncol
                psv[pl.ds(g, L)] = _sqrt16(psum)
                nsv[pl.ds(g, L)] = _sqrt16(nsum)

            pltpu.sync_copy(psv, pos_out.at[pl.ds(base, CH)])
            pltpu.sync_copy(nsv, neg_out.at[pl.ds(base, CH)])

    return trans_e


_sc_trans_e = _make_sc_kernel()


def kernel(pos_edge_index, neg_edge_index, edge_type, entity_table, relation_table):
    ph = pos_edge_index[0].astype(jnp.int32)
    pt = pos_edge_index[1].astype(jnp.int32)
    nh = neg_edge_index[0].astype(jnp.int32)
    nt = neg_edge_index[1].astype(jnp.int32)
    et = edge_type.astype(jnp.int32)
    return _sc_trans_e(ph, pt, nh, nt, et, entity_table, relation_table)


# speedup vs baseline: 1.0561x; 1.0561x over previous
import dataclasses
import functools

import jax
import jax.numpy as jnp
from jax import lax
from jax.experimental import pallas as pl
from jax.experimental.pallas import tpu as pltpu
from jax.experimental.pallas import tpu_sc as plsc

B = 16384
D = 64
NC = 2
NS = 16
NW = NC * NS
BPW = B // NW
CH = 128
NCHUNK = BPW // CH
L = 16
G = 16


def _sqrt16(x):
    x = jnp.maximum(x, jnp.float32(1e-30))
    xi = lax.bitcast_convert_type(x, jnp.int32)
    yi = (xi >> 1) + jnp.int32(0x1FBD1DF5)
    y = lax.bitcast_convert_type(yi, jnp.float32)
    half = jnp.float32(0.5)
    y = half * (y + x / y)
    y = half * (y + x / y)
    y = half * (y + x / y)
    return y


def _make_sc_kernel():
    mesh = plsc.VectorSubcoreMesh(core_axis_name="c", subcore_axis_name="s")
    out_sds = jax.ShapeDtypeStruct((B,), jnp.float32)
    cp = pltpu.CompilerParams()
    if "needs_layout_passes" in pltpu.CompilerParams.__dataclass_fields__:
        cp = dataclasses.replace(cp, needs_layout_passes=False)
    if "use_tc_tiling_on_sc" in pltpu.CompilerParams.__dataclass_fields__:
        cp = dataclasses.replace(cp, use_tc_tiling_on_sc=True)

    @functools.partial(
        pl.kernel,
        mesh=mesh,
        out_type=(out_sds, out_sds),
        compiler_params=cp,
        scratch_types=[
            pltpu.VMEM((CH,), jnp.int32),
            pltpu.VMEM((CH,), jnp.int32),
            pltpu.VMEM((CH,), jnp.int32),
            pltpu.VMEM((CH,), jnp.int32),
            pltpu.VMEM((CH,), jnp.int32),
            pltpu.VMEM((CH, D), jnp.float32),
            pltpu.VMEM((CH, D), jnp.float32),
            pltpu.VMEM((CH, D), jnp.float32),
            pltpu.VMEM((CH, D), jnp.float32),
            pltpu.VMEM((CH, D), jnp.float32),
            pltpu.VMEM((L, L), jnp.float32),
            pltpu.VMEM((L, L), jnp.float32),
            pltpu.VMEM((CH,), jnp.float32),
            pltpu.VMEM((CH,), jnp.float32),
            pltpu.SemaphoreType.DMA,
            pltpu.SemaphoreType.DMA,
        ],
    )
    def trans_e(ph_hbm, pt_hbm, nh_hbm, nt_hbm, et_hbm, ent_hbm, rel_hbm,
                pos_out, neg_out,
                phv, ptv, nhv, ntv, etv, hv, tv, gv, uv, rv,
                mp, mn, psv, nsv, semA, semB):
        wid = lax.axis_index("s") * NC + lax.axis_index("c")
        row_iota = lax.iota(jnp.int32, L)
        streams = ((phv, hv, ent_hbm), (ptv, tv, ent_hbm),
                   (nhv, gv, ent_hbm), (ntv, uv, ent_hbm),
                   (etv, rv, rel_hbm))

        def fire(g, sem):
            for (iv, buf, table) in streams:
                ivec = iv[pl.ds(g, G)]
                for rr in range(G):
                    pltpu.async_copy(table.at[ivec[rr]], buf.at[g + rr], sem)

        def drain(g, sem):
            for (_, buf, table) in streams:
                pltpu.make_async_copy(
                    table.at[pl.ds(0, G)], buf.at[pl.ds(g, G)], sem).wait()

        def compute(g):
            for rr in range(G):
                row = g + rr
                pacc = None
                nacc = None
                for j in range(D // L):
                    sl = pl.ds(j * L, L)
                    rel16 = rv[row, sl]
                    dp = hv[row, sl] + rel16 - tv[row, sl]
                    dn = gv[row, sl] + rel16 - uv[row, sl]
                    pacc = dp * dp if pacc is None else pacc + dp * dp
                    nacc = dn * dn if nacc is None else nacc + dn * dn
                mp[rr, pl.ds(0, L)] = pacc
                mn[rr, pl.ds(0, L)] = nacc
            psum = None
            nsum = None
            for col in range(L):
                ci = jnp.full((L,), col, jnp.int32)
                pc = plsc.load_gather(mp, [row_iota, ci])
                ncol = plsc.load_gather(mn, [row_iota, ci])
                psum = pc if psum is None else psum + pc
                nsum = ncol if nsum is None else nsum + ncol
            psv[pl.ds(g, L)] = _sqrt16(psum)
            nsv[pl.ds(g, L)] = _sqrt16(nsum)

        @pl.loop(0, NCHUNK)
        def _(c):
            base = wid * BPW + c * CH
            pltpu.sync_copy(ph_hbm.at[pl.ds(base, CH)], phv)
            pltpu.sync_copy(pt_hbm.at[pl.ds(base, CH)], ptv)
            pltpu.sync_copy(nh_hbm.at[pl.ds(base, CH)], nhv)
            pltpu.sync_copy(nt_hbm.at[pl.ds(base, CH)], ntv)
            pltpu.sync_copy(et_hbm.at[pl.ds(base, CH)], etv)

            fire(0, semA)

            @pl.loop(0, CH, step=2 * G)
            def _(g):
                fire(g + G, semB)
                drain(g, semA)
                compute(g)

                @pl.when(g + 2 * G < CH)
                def _():
                    fire(g + 2 * G, semA)

                drain(g + G, semB)
                compute(g + G)

            pltpu.sync_copy(psv, pos_out.at[pl.ds(base, CH)])
            pltpu.sync_copy(nsv, neg_out.at[pl.ds(base, CH)])

    return trans_e


_sc_trans_e = _make_sc_kernel()


def kernel(pos_edge_index, neg_edge_index, edge_type, entity_table, relation_table):
    ph = pos_edge_index[0].astype(jnp.int32)
    pt = pos_edge_index[1].astype(jnp.int32)
    nh = neg_edge_index[0].astype(jnp.int32)
    nt = neg_edge_index[1].astype(jnp.int32)
    et = edge_type.astype(jnp.int32)
    return _sc_trans_e(ph, pt, nh, nt, et, entity_table, relation_table)
